# baseline (speedup 1.0000x reference)
import functools

import jax
import jax.numpy as jnp
from jax import lax
from jax.experimental import pallas as pl
from jax.experimental.pallas import tpu as pltpu
from jax.experimental.pallas import tpu_sc as plsc

EPS = 1e-06


def _sc_gather(table, idx, n_tokens, mm_dim, nw, chunk):
    n_chunks = n_tokens // (nw * chunk)
    mesh = plsc.VectorSubcoreMesh(core_axis_name="c", subcore_axis_name="s")
    nc = mesh.num_cores

    @functools.partial(
        pl.kernel,
        out_type=jax.ShapeDtypeStruct((n_tokens, mm_dim), jnp.float32),
        mesh=mesh,
        scratch_types=[
            pltpu.VMEM((n_chunks, chunk), jnp.int32),
            pltpu.VMEM((chunk, mm_dim), jnp.float32),
            pltpu.VMEM((chunk, mm_dim), jnp.float32),
            pltpu.VMEM((chunk, mm_dim), jnp.float32),
            pltpu.SemaphoreType.DMA,
            pltpu.SemaphoreType.DMA,
        ],
    )
    def gather_kernel(table_hbm, idx_hbm, out_hbm,
                      idx_v, rows_a, rows_b, rows_c, g_sem, o_sem):
        wid = lax.axis_index("s") * nc + lax.axis_index("c")
        base = wid * n_chunks * chunk
        bufs = (rows_a, rows_b, rows_c)

        def gather(c):
            return pltpu.make_async_copy(
                table_hbm.at[idx_v.at[c]], bufs[c % 3], g_sem)

        def copyout(c):
            return pltpu.make_async_copy(
                bufs[c % 3], out_hbm.at[pl.ds(base + c * chunk, chunk)], o_sem)

        pltpu.sync_copy(idx_hbm.at[wid], idx_v)
        gather(0).start()
        gather(1).start()
        for c in range(n_chunks):
            gather(c).wait()
            copyout(c).start()
            g = c + 2
            if g < n_chunks:
                if g >= 3:
                    copyout(g - 3).wait()
                gather(g).start()
        for c in range(n_chunks - 3, n_chunks):
            copyout(c).wait()

    return gather_kernel(table, idx)


def _tc_norm_proj_norm(emb, w_bf16, n_tokens, mm_dim, txt_dim, blk):
    def body(x_ref, w_ref, o_ref):
        x = x_ref[...]
        m2 = jnp.mean(x * x, axis=-1, keepdims=True) + EPS
        z = lax.dot_general(
            x.astype(jnp.bfloat16), w_ref[...], (((1,), (1,)), ((), ())),
            preferred_element_type=jnp.float32,
        )
        mz = jnp.mean(z * z, axis=-1, keepdims=True)
        o_ref[...] = z * lax.rsqrt(mz + EPS * m2)

    return pl.pallas_call(
        body,
        grid=(n_tokens // blk,),
        in_specs=[
            pl.BlockSpec((blk, mm_dim), lambda i: (i, 0)),
            pl.BlockSpec((txt_dim, mm_dim), lambda i: (0, 0)),
        ],
        out_specs=pl.BlockSpec((blk, txt_dim), lambda i: (i, 0)),
        out_shape=jax.ShapeDtypeStruct((n_tokens, txt_dim), jnp.float32),
    )(emb, w_bf16)


def kernel(input_ids, embedding_table, hard_norm_weight, proj_weight):
    b, s = input_ids.shape
    vocab, mm_dim = embedding_table.shape
    txt_dim = proj_weight.shape[0]
    n_tokens = b * s

    nw = 32
    chunk = 16
    w_eff = (proj_weight * hard_norm_weight).astype(jnp.bfloat16)
    ids = input_ids.reshape(nw, n_tokens // (nw * chunk), chunk).astype(jnp.int32)

    emb = _sc_gather(embedding_table, ids, n_tokens, mm_dim, nw, chunk)
    out = _tc_norm_proj_norm(emb, w_eff, n_tokens, mm_dim, txt_dim, blk=1024)
    return out.reshape(b, s, txt_dim)


# speedup vs baseline: 1.0003x; 1.0003x over previous
import functools

import jax
import jax.numpy as jnp
from jax import lax
from jax.experimental import pallas as pl
from jax.experimental.pallas import tpu as pltpu
from jax.experimental.pallas import tpu_sc as plsc

EPS = 1e-06


def _sc_gather(table, idx, n_tokens, mm_dim, nw, chunk):
    n_chunks = n_tokens // (nw * chunk)
    mesh = plsc.VectorSubcoreMesh(core_axis_name="c", subcore_axis_name="s")
    nc = mesh.num_cores

    @functools.partial(
        pl.kernel,
        out_type=jax.ShapeDtypeStruct((n_tokens, mm_dim), jnp.float32),
        mesh=mesh,
        scratch_types=[
            pltpu.VMEM((n_chunks, chunk), jnp.int32),
            pltpu.VMEM((chunk, mm_dim), jnp.float32),
            pltpu.VMEM((chunk, mm_dim), jnp.float32),
            pltpu.VMEM((chunk, mm_dim), jnp.float32),
            pltpu.VMEM((chunk, mm_dim), jnp.float32),
            pltpu.SemaphoreType.DMA,
            pltpu.SemaphoreType.DMA,
        ],
    )
    def gather_kernel(table_hbm, idx_hbm, out_hbm,
                      idx_v, rows_a, rows_b, rows_c, rows_d, g_sem, o_sem):
        wid = lax.axis_index("s") * nc + lax.axis_index("c")
        base = wid * n_chunks * chunk
        bufs = (rows_a, rows_b, rows_c, rows_d)
        depth = len(bufs)

        def gather(c):
            return pltpu.make_async_copy(
                table_hbm.at[idx_v.at[c]], bufs[c % depth], g_sem)

        def copyout(c):
            return pltpu.make_async_copy(
                bufs[c % depth], out_hbm.at[pl.ds(base + c * chunk, chunk)],
                o_sem)

        pltpu.sync_copy(idx_hbm.at[wid], idx_v)
        for c in range(depth - 1):
            gather(c).start()
        for c in range(n_chunks):
            gather(c).wait()
            copyout(c).start()
            g = c + depth - 1
            if g < n_chunks:
                if g >= depth:
                    copyout(g - depth).wait()
                gather(g).start()
        for c in range(n_chunks - depth, n_chunks):
            copyout(c).wait()

    return gather_kernel(table, idx)


def _tc_norm_proj_norm(emb, w_bf16, n_tokens, mm_dim, txt_dim, blk):
    def body(x_ref, w_ref, o_ref):
        x = x_ref[...]
        m2 = jnp.mean(x * x, axis=-1, keepdims=True) + EPS
        z = lax.dot_general(
            x.astype(jnp.bfloat16), w_ref[...], (((1,), (1,)), ((), ())),
            preferred_element_type=jnp.float32,
        )
        mz = jnp.mean(z * z, axis=-1, keepdims=True)
        o_ref[...] = z * lax.rsqrt(mz + EPS * m2)

    return pl.pallas_call(
        body,
        grid=(n_tokens // blk,),
        in_specs=[
            pl.BlockSpec((blk, mm_dim), lambda i: (i, 0)),
            pl.BlockSpec((txt_dim, mm_dim), lambda i: (0, 0)),
        ],
        out_specs=pl.BlockSpec((blk, txt_dim), lambda i: (i, 0)),
        out_shape=jax.ShapeDtypeStruct((n_tokens, txt_dim), jnp.float32),
    )(emb, w_bf16)


def kernel(input_ids, embedding_table, hard_norm_weight, proj_weight):
    b, s = input_ids.shape
    vocab, mm_dim = embedding_table.shape
    txt_dim = proj_weight.shape[0]
    n_tokens = b * s

    nw = 32
    chunk = 16
    w_eff = (proj_weight * hard_norm_weight).astype(jnp.bfloat16)
    ids = input_ids.reshape(nw, n_tokens // (nw * chunk), chunk).astype(jnp.int32)

    emb = _sc_gather(embedding_table, ids, n_tokens, mm_dim, nw, chunk)
    out = _tc_norm_proj_norm(emb, w_eff, n_tokens, mm_dim, txt_dim, blk=1024)
    return out.reshape(b, s, txt_dim)
